# baseline (speedup 1.0000x reference)
import functools

import jax
import jax.numpy as jnp
from jax import lax
from jax.experimental import pallas as pl
from jax.experimental.pallas import tpu as pltpu
from jax.experimental.pallas import tpu_sc as plsc

_N, _C, _A = 1024, 1000, 64

_NC, _NS, _L = 2, 16, 16
_NW = _NC * _NS
_BPW = _N // _NW


def _sc_gather_body(tbl_hbm, lab_hbm,
                    wc_hbm,
                    idx_v, rows_v, sem1):
    wid = lax.axis_index("s") * _NC + lax.axis_index("c")
    base = wid * _BPW
    pltpu.sync_copy(lab_hbm.at[pl.ds(base, _BPW)], idx_v)
    pltpu.async_copy(tbl_hbm.at[idx_v], rows_v, sem1).wait()
    pltpu.sync_copy(rows_v, wc_hbm.at[pl.ds(base, _BPW)])


@functools.cache
def _sc_gather():
    return pl.kernel(
        _sc_gather_body,
        mesh=plsc.VectorSubcoreMesh(core_axis_name="c", subcore_axis_name="s"),
        compiler_params=pltpu.CompilerParams(skip_device_barrier=True),
        out_type=jax.ShapeDtypeStruct((_N, 2 * _A), jnp.float32),
        scratch_types=[
            pltpu.VMEM((_BPW,), jnp.int32),
            pltpu.VMEM((_BPW, 2 * _A), jnp.float32),
            pltpu.SemaphoreType.DMA,
        ],
    )


def _tc_loss_body(ratio_ref, fc_ref, ys_ref, lab_ref, wc_hbm,
                  wts_ref, out_ref, wc_vmem, dma_sem):
    cp = pltpu.make_async_copy(wc_hbm, wc_vmem, dma_sem)
    cp.start()
    fc = fc_ref[...]
    ys = ys_ref[...]
    cp.wait()
    wc = wc_vmem[...]
    w = wc[:, :_A]
    cvt = wc[:, _A:]
    ratio = ratio_ref[0]

    dn = (((1,), (1,)), ((), ()))
    t1 = lax.dot_general(cvt, fc * fc, dn,
                         preferred_element_type=jnp.float32)
    t2 = lax.dot_general(w * cvt, fc, dn,
                         preferred_element_type=jnp.float32)
    const = jnp.sum(w * w * cvt, axis=1, keepdims=True)
    aug = ys + (0.5 * ratio) * (t1 - 2.0 * t2 + const)

    m = jnp.max(aug, axis=1, keepdims=True)
    lse = jnp.log(jnp.sum(jnp.exp(aug - m), axis=1, keepdims=True)) + m

    lab = jnp.reshape(lab_ref[...], (_N, 1))
    iota = lax.broadcasted_iota(jnp.int32, ys.shape, 1)
    onehot = iota == lab
    ysl = jnp.sum(jnp.where(onehot, ys, 0.0), axis=1, keepdims=True)
    wrow = jnp.reshape(wts_ref[...], (1, _C))
    wl = jnp.sum(jnp.where(onehot, wrow, 0.0), axis=1, keepdims=True)

    nll = lse - ysl
    out_ref[0] = jnp.sum(wl * nll) / jnp.sum(wl)


def _tc_loss(ratio, fc, y_s, lab, wc, wts, interpret=False):
    return pl.pallas_call(
        _tc_loss_body,
        out_shape=jax.ShapeDtypeStruct((1,), jnp.float32),
        in_specs=[
            pl.BlockSpec(memory_space=pltpu.SMEM),
            pl.BlockSpec(memory_space=pltpu.VMEM),
            pl.BlockSpec(memory_space=pltpu.VMEM),
            pl.BlockSpec(memory_space=pltpu.VMEM),
            pl.BlockSpec(memory_space=pl.ANY),
            pl.BlockSpec(memory_space=pltpu.VMEM),
        ],
        out_specs=pl.BlockSpec(memory_space=pltpu.SMEM),
        scratch_shapes=[
            pltpu.VMEM((_N, 2 * _A), jnp.float32),
            pltpu.SemaphoreType.DMA,
        ],
        compiler_params=pltpu.CompilerParams(skip_device_barrier=True),
        interpret=interpret,
    )(ratio, fc, y_s, lab, wc, wts)


def kernel(fc, features_source, y_s, labels_source, ratio, weights, cv, mode):
    tbl = jnp.concatenate([fc, cv], axis=1)
    wc = _sc_gather()(tbl, labels_source)
    ratio1 = jnp.reshape(ratio, (1,)).astype(jnp.float32)
    loss = _tc_loss(ratio1, fc, y_s, labels_source, wc, weights)
    return loss[0]


# speedup vs baseline: 1.0636x; 1.0636x over previous
import functools

import jax
import jax.numpy as jnp
from jax import lax
from jax.experimental import pallas as pl
from jax.experimental.pallas import tpu as pltpu
from jax.experimental.pallas import tpu_sc as plsc

_N, _C, _A = 1024, 1000, 64

_NC, _NS, _L = 2, 16, 16
_NW = _NC * _NS
_BPW = _N // _NW


def _sc_gather_body(tbl_hbm, lab_hbm,
                    wc_hbm,
                    idx_v, rows_v, sem1):
    wid = lax.axis_index("s") * _NC + lax.axis_index("c")
    base = wid * _BPW
    pltpu.sync_copy(lab_hbm.at[pl.ds(base, _BPW)], idx_v)
    pltpu.async_copy(tbl_hbm.at[idx_v], rows_v, sem1).wait()
    pltpu.sync_copy(rows_v, wc_hbm.at[pl.ds(base, _BPW)])


@functools.cache
def _sc_gather():
    return pl.kernel(
        _sc_gather_body,
        mesh=plsc.VectorSubcoreMesh(core_axis_name="c", subcore_axis_name="s"),
        compiler_params=pltpu.CompilerParams(skip_device_barrier=True),
        out_type=jax.ShapeDtypeStruct((_N, 2 * _A), jnp.float32),
        scratch_types=[
            pltpu.VMEM((_BPW,), jnp.int32),
            pltpu.VMEM((_BPW, 2 * _A), jnp.float32),
            pltpu.SemaphoreType.DMA,
        ],
    )


def _tc_loss_body(ratio_ref, fc_ref, ysT_ref, lab_ref, wc_ref,
                  wts_ref, out_ref):
    fc = fc_ref[...]
    ysT = ysT_ref[...]
    wc = wc_ref[...]
    w = wc[:, :_A]
    cvt = wc[:, _A:]
    ratio = ratio_ref[0]

    dn = (((1,), (1,)), ((), ()))
    t1 = lax.dot_general(fc * fc, cvt, dn,
                         preferred_element_type=jnp.float32)
    t2 = lax.dot_general(fc, w * cvt, dn,
                         preferred_element_type=jnp.float32)
    ones_a = jnp.ones((1, _A), jnp.float32)
    const = lax.dot_general(ones_a, w * w * cvt, dn,
                            preferred_element_type=jnp.float32)
    augT = ysT + (0.5 * ratio) * (t1 - 2.0 * t2 + const)

    m = jnp.max(augT, axis=0, keepdims=True)
    lse = jnp.log(jnp.sum(jnp.exp(augT - m), axis=0, keepdims=True)) + m

    lab = jnp.reshape(lab_ref[...], (1, _N))
    iota = lax.broadcasted_iota(jnp.int32, ysT.shape, 0)
    onehotT = iota == lab
    ysl = jnp.sum(jnp.where(onehotT, ysT, 0.0), axis=0, keepdims=True)
    wrow = jnp.reshape(wts_ref[...], (1, _C))
    dn0 = (((1,), (0,)), ((), ()))
    wl = lax.dot_general(wrow, onehotT.astype(jnp.float32), dn0,
                         preferred_element_type=jnp.float32)

    nll = lse - ysl
    out_ref[0] = jnp.sum(wl * nll) / jnp.sum(wl)


def _tc_loss(ratio, fc, ysT, lab, wc, wts, interpret=False):
    return pl.pallas_call(
        _tc_loss_body,
        out_shape=jax.ShapeDtypeStruct((1,), jnp.float32),
        in_specs=[
            pl.BlockSpec(memory_space=pltpu.SMEM),
            pl.BlockSpec(memory_space=pltpu.VMEM),
            pl.BlockSpec(memory_space=pltpu.VMEM),
            pl.BlockSpec(memory_space=pltpu.VMEM),
            pl.BlockSpec(memory_space=pltpu.VMEM),
            pl.BlockSpec(memory_space=pltpu.VMEM),
        ],
        out_specs=pl.BlockSpec(memory_space=pltpu.SMEM),
        compiler_params=pltpu.CompilerParams(skip_device_barrier=True),
        interpret=interpret,
    )(ratio, fc, ysT, lab, wc, wts)


def kernel(fc, features_source, y_s, labels_source, ratio, weights, cv, mode):
    tbl = jnp.concatenate([fc, cv], axis=1)
    wc = _sc_gather()(tbl, labels_source)
    ratio1 = jnp.reshape(ratio, (1,)).astype(jnp.float32)
    loss = _tc_loss(ratio1, fc, jnp.transpose(y_s), labels_source, wc,
                    weights)
    return loss[0]


# speedup vs baseline: 1.1053x; 1.0392x over previous
import functools

import jax
import jax.numpy as jnp
from jax import lax
from jax.experimental import pallas as pl
from jax.experimental.pallas import tpu as pltpu
from jax.experimental.pallas import tpu_sc as plsc

_N, _C, _A = 1024, 1000, 64

_NC, _NS, _L = 2, 16, 16
_NW = _NC * _NS
_BPW = _N // _NW


def _sc_gather_body(tbl_hbm, lab_hbm,
                    wc_hbm,
                    idx_v, rows_v, sem1):
    wid = lax.axis_index("s") * _NC + lax.axis_index("c")
    base = wid * _BPW
    pltpu.sync_copy(lab_hbm.at[pl.ds(base, _BPW)], idx_v)
    pltpu.async_copy(tbl_hbm.at[idx_v], rows_v, sem1).wait()
    pltpu.sync_copy(rows_v, wc_hbm.at[pl.ds(base, _BPW)])


@functools.cache
def _sc_gather():
    return pl.kernel(
        _sc_gather_body,
        mesh=plsc.VectorSubcoreMesh(core_axis_name="c", subcore_axis_name="s"),
        compiler_params=pltpu.CompilerParams(skip_device_barrier=True),
        out_type=jax.ShapeDtypeStruct((_N, 2 * _A), jnp.float32),
        scratch_types=[
            pltpu.VMEM((_BPW,), jnp.int32),
            pltpu.VMEM((_BPW, 2 * _A), jnp.float32),
            pltpu.SemaphoreType.DMA,
        ],
    )


def _tc_loss_body(ratio_ref, fc_ref, ysT_ref, lab_ref, wc_ref,
                  wts_ref, out_ref):
    fc = fc_ref[...]
    ysT = ysT_ref[...]
    wc = wc_ref[...]
    w = wc[:, :_A]
    cvt = wc[:, _A:]
    ratio = ratio_ref[0]

    dn = (((1,), (1,)), ((), ()))
    lhs = jnp.concatenate([fc * fc, fc], axis=1)
    wcvt = w * cvt
    rhs = jnp.concatenate([cvt, -2.0 * wcvt], axis=1)
    t12 = lax.dot_general(lhs, rhs, dn,
                          preferred_element_type=jnp.float32)
    ones_a = jnp.ones((1, _A), jnp.float32)
    const = lax.dot_general(ones_a, w * wcvt, dn,
                            preferred_element_type=jnp.float32)
    augT = ysT + (0.5 * ratio) * (t12 + const)

    m = jnp.max(augT, axis=0, keepdims=True)
    lse = jnp.log(jnp.sum(jnp.exp(augT - m), axis=0, keepdims=True)) + m

    lab = jnp.reshape(lab_ref[...], (1, _N))
    iota = lax.broadcasted_iota(jnp.int32, ysT.shape, 0)
    onehotT = iota == lab
    ysl = jnp.sum(jnp.where(onehotT, ysT, 0.0), axis=0, keepdims=True)
    wrow = jnp.reshape(wts_ref[...], (1, _C))
    dn0 = (((1,), (0,)), ((), ()))
    wl = lax.dot_general(wrow, onehotT.astype(jnp.float32), dn0,
                         preferred_element_type=jnp.float32)

    nll = lse - ysl
    out_ref[0] = jnp.sum(wl * nll) / jnp.sum(wl)


def _tc_loss(ratio, fc, ysT, lab, wc, wts, interpret=False):
    return pl.pallas_call(
        _tc_loss_body,
        out_shape=jax.ShapeDtypeStruct((1,), jnp.float32),
        in_specs=[
            pl.BlockSpec(memory_space=pltpu.SMEM),
            pl.BlockSpec(memory_space=pltpu.VMEM),
            pl.BlockSpec(memory_space=pltpu.VMEM),
            pl.BlockSpec(memory_space=pltpu.VMEM),
            pl.BlockSpec(memory_space=pltpu.VMEM),
            pl.BlockSpec(memory_space=pltpu.VMEM),
        ],
        out_specs=pl.BlockSpec(memory_space=pltpu.SMEM),
        compiler_params=pltpu.CompilerParams(skip_device_barrier=True),
        interpret=interpret,
    )(ratio, fc, ysT, lab, wc, wts)


def kernel(fc, features_source, y_s, labels_source, ratio, weights, cv, mode):
    tbl = jnp.concatenate([fc, cv], axis=1)
    wc = _sc_gather()(tbl, labels_source)
    ratio1 = jnp.reshape(ratio, (1,)).astype(jnp.float32)
    loss = _tc_loss(ratio1, fc, jnp.transpose(y_s), labels_source, wc,
                    weights)
    return loss[0]
